# Initial kernel scaffold; baseline (speedup 1.0000x reference)
import jax
import jax.numpy as jnp
from jax.experimental import pallas as pl


def kernel(all_users, all_items, edge_index, edge_weight):
    raise NotImplementedError("write your pallas kernel here")


# speedup vs baseline: 3.7048x; 3.7048x over previous
import functools

import jax
import jax.numpy as jnp
from jax import lax
from jax.experimental import pallas as pl
from jax.experimental.pallas import tpu as pltpu
from jax.experimental.pallas import tpu_sc as plsc

_D = 64
_N = 50000
_HALF = 25088
_NPAD = 2 * _HALF
_GARB = 16
_ACC = _HALF + _GARB
_EPAD = 802816
_IDXROWS = _EPAD // 128
_TILES = 16
_RPT = _IDXROWS // _TILES
_K = 2
_CHUNK_E = _K * 128
_NCH = _RPT // _K
_CPT = _HALF // _TILES


def _layer_body(emb, src, dst, w, out, src_v, dst_v, dloc_v, w_v, rows_v, acc,
                sem):
    c = lax.axis_index("c")
    s = lax.axis_index("s")
    half_base = c * _HALF
    lane = lax.iota(jnp.int32, 16)
    zv = jnp.zeros((16,), jnp.float32)

    def _zrow(i, _):
        for b in range(4):
            rows_v[i, pl.ds(b * 16, 16)] = zv
        return 0

    lax.fori_loop(0, _CHUNK_E, _zrow, 0)
    lb = s * _CPT
    for t in range(_CPT // _CHUNK_E):
        pltpu.sync_copy(rows_v, acc.at[pl.ds(lb + t * _CHUNK_E, _CHUNK_E)])
    if _CPT % _CHUNK_E:
        pltpu.sync_copy(rows_v.at[pl.ds(0, _CPT % _CHUNK_E)],
                        acc.at[pl.ds(lb + _CPT - _CPT % _CHUNK_E,
                                     _CPT % _CHUNK_E)])

    @pl.when(s == 0)
    def _zero_garbage():
        pltpu.sync_copy(rows_v.at[pl.ds(0, _GARB)], acc.at[pl.ds(_HALF, _GARB)])

    plsc.subcore_barrier()

    row0 = s * _RPT

    def _chunk(ci, _):
        base = row0 + ci * _K
        pltpu.sync_copy(src.at[pl.ds(base, _K)], src_v)
        pltpu.sync_copy(dst.at[pl.ds(base, _K)], dst_v)
        pltpu.sync_copy(w.at[pl.ds(base * 128, _CHUNK_E)], w_v)
        cps = []
        for j in range(_K):
            cps.append(pltpu.async_copy(emb.at[src_v.at[j]],
                                        rows_v.at[pl.ds(j * 128, 128)], sem))
        for j in range(_K):
            for i in range(8):
                dv = dst_v[j, pl.ds(i * 16, 16)]
                loc = dv - half_base
                ok = (loc >= 0) & (loc < _HALF)
                loc = jnp.where(ok, loc, _HALF + lane)
                dloc_v[j, pl.ds(i * 16, 16)] = loc
        for cp in cps:
            cp.wait()

        def _scale(g, _):
            e0 = g * 16
            wv16 = w_v[pl.ds(e0, 16)]
            for k in range(16):
                wk = jnp.full((16,), wv16[k], jnp.float32)
                e = e0 + k
                for b in range(4):
                    rows_v[e, pl.ds(b * 16, 16)] = (
                        rows_v[e, pl.ds(b * 16, 16)] * wk)
            return 0

        lax.fori_loop(0, _CHUNK_E // 16, _scale, 0)
        for j in range(_K):
            pltpu.sync_copy(rows_v.at[pl.ds(j * 128, 128)],
                            acc.at[dloc_v.at[j]], add=True)
        return 0

    lax.fori_loop(0, _NCH, _chunk, 0)

    plsc.subcore_barrier()
    pltpu.sync_copy(acc.at[pl.ds(lb, _CPT)],
                    out.at[pl.ds(half_base + lb, _CPT)])


_layer = functools.partial(
    pl.kernel,
    mesh=plsc.VectorSubcoreMesh(core_axis_name="c", subcore_axis_name="s"),
    out_type=jax.ShapeDtypeStruct((_NPAD, _D), jnp.float32),
    compiler_params=pltpu.CompilerParams(use_tc_tiling_on_sc=False),
    scratch_types=[
        pltpu.VMEM((_K, 128), jnp.int32),
        pltpu.VMEM((_K, 128), jnp.int32),
        pltpu.VMEM((_K, 128), jnp.int32),
        pltpu.VMEM((_CHUNK_E,), jnp.float32),
        pltpu.VMEM((_CHUNK_E, _D), jnp.float32),
        pltpu.VMEM_SHARED((_ACC, _D), jnp.float32),
        pltpu.SemaphoreType.DMA,
    ],
)(_layer_body)


def _mean_body(a, b, c, d, o):
    o[...] = (a[...] + b[...] + c[...] + d[...]) * 0.25


def _mean4(e0, e1, e2, e3):
    bs = pl.BlockSpec((1024, _D), lambda i: (i, 0))
    return pl.pallas_call(
        _mean_body,
        grid=(_NPAD // 1024,),
        in_specs=[bs] * 4,
        out_specs=bs,
        out_shape=jax.ShapeDtypeStruct((_NPAD, _D), jnp.float32),
    )(e0, e1, e2, e3)


def kernel(all_users, all_items, edge_index, edge_weight):
    n_users = all_users.shape[0]
    emb0 = jnp.concatenate([all_users, all_items], axis=0)
    emb0 = jnp.pad(emb0, ((0, _NPAD - _N), (0, 0)))
    src = edge_index[0].astype(jnp.int32)
    dst = edge_index[1].astype(jnp.int32)
    w = edge_weight.astype(jnp.float32)
    pad = _EPAD - src.shape[0]
    src = jnp.pad(src, (0, pad)).reshape(_IDXROWS, 128)
    dst = jnp.pad(dst, (0, pad), constant_values=_NPAD).reshape(_IDXROWS, 128)
    w = jnp.pad(w, (0, pad))
    e1 = _layer(emb0, src, dst, w)
    e2 = _layer(e1, src, dst, w)
    e3 = _layer(e2, src, dst, w)
    out = _mean4(emb0, e1, e2, e3)
    return (out[:n_users], out[n_users:_N])


# speedup vs baseline: 5.6219x; 1.5175x over previous
import functools

import jax
import jax.numpy as jnp
from jax import lax
from jax.experimental import pallas as pl
from jax.experimental.pallas import tpu as pltpu
from jax.experimental.pallas import tpu_sc as plsc

_D = 64
_N = 50000
_HALF = 25088
_NPAD = 2 * _HALF
_GARB = 16
_ACC = _HALF + _GARB
_EPAD = 802816
_IDXROWS = _EPAD // 128
_TILES = 16
_RPT = _IDXROWS // _TILES
_K = 8
_BATCH_E = _K * 128
_NB = _RPT // _K
_CPT = _HALF // _TILES


def _layer_body(emb, src, dst, w, out, src_v, dst_v, dloc_v, w_v, rows_a,
                rows_b, acc, sem):
    c = lax.axis_index("c")
    s = lax.axis_index("s")
    half_base = c * _HALF
    lane = lax.iota(jnp.int32, 16)
    zv = jnp.zeros((16,), jnp.float32)

    def _zrow(i, _):
        for b in range(4):
            rows_a[i, pl.ds(b * 16, 16)] = zv
        return 0

    lax.fori_loop(0, 128, _zrow, 0)
    lb = s * _CPT
    for t in range(_CPT // 128):
        pltpu.sync_copy(rows_a, acc.at[pl.ds(lb + t * 128, 128)])
    if _CPT % 128:
        pltpu.sync_copy(rows_a.at[pl.ds(0, _CPT % 128)],
                        acc.at[pl.ds(lb + _CPT - _CPT % 128, _CPT % 128)])

    @pl.when(s == 0)
    def _zero_garbage():
        pltpu.sync_copy(rows_a.at[pl.ds(0, _GARB)], acc.at[pl.ds(_HALF, _GARB)])

    plsc.subcore_barrier()

    row0 = s * _RPT
    bufs = (rows_a, rows_b)

    def _scale(buf, j):
        def _grp(g, _):
            wv16 = w_v[pl.ds(j * 128 + g * 16, 16)]
            e0 = g * 16
            for k in range(16):
                wk = jnp.full((16,), wv16[k], jnp.float32)
                for b in range(4):
                    buf[e0 + k, pl.ds(b * 16, 16)] = (
                        buf[e0 + k, pl.ds(b * 16, 16)] * wk)
            return 0

        lax.fori_loop(0, 8, _grp, 0)

    def _batch(ci, _):
        base = row0 + ci * _K
        pltpu.sync_copy(src.at[pl.ds(base, _K)], src_v)
        pltpu.sync_copy(dst.at[pl.ds(base, _K)], dst_v)
        pltpu.sync_copy(w.at[pl.ds(base * 128, _BATCH_E)], w_v)
        for j in range(_K):
            for i in range(8):
                dv = dst_v[j, pl.ds(i * 16, 16)]
                loc = dv - half_base
                ok = (loc >= 0) & (loc < _HALF)
                loc = jnp.where(ok, loc, _HALF + lane)
                dloc_v[j, pl.ds(i * 16, 16)] = loc
        cp = pltpu.async_copy(emb.at[src_v.at[0]], bufs[0], sem)
        for j in range(_K):
            buf = bufs[j % 2]
            nxt = bufs[(j + 1) % 2]
            if j + 1 < _K:
                cp_next = pltpu.async_copy(emb.at[src_v.at[j + 1]], nxt, sem)
            cp.wait()
            _scale(buf, j)
            pltpu.sync_copy(buf, acc.at[dloc_v.at[j]], add=True)
            if j + 1 < _K:
                cp = cp_next
        return 0

    lax.fori_loop(0, _NB, _batch, 0)

    plsc.subcore_barrier()
    pltpu.sync_copy(acc.at[pl.ds(lb, _CPT)],
                    out.at[pl.ds(half_base + lb, _CPT)])


_layer = functools.partial(
    pl.kernel,
    mesh=plsc.VectorSubcoreMesh(core_axis_name="c", subcore_axis_name="s"),
    out_type=jax.ShapeDtypeStruct((_NPAD, _D), jnp.float32),
    compiler_params=pltpu.CompilerParams(use_tc_tiling_on_sc=False),
    scratch_types=[
        pltpu.VMEM((_K, 128), jnp.int32),
        pltpu.VMEM((_K, 128), jnp.int32),
        pltpu.VMEM((_K, 128), jnp.int32),
        pltpu.VMEM((_BATCH_E,), jnp.float32),
        pltpu.VMEM((128, _D), jnp.float32),
        pltpu.VMEM((128, _D), jnp.float32),
        pltpu.VMEM_SHARED((_ACC, _D), jnp.float32),
        pltpu.SemaphoreType.DMA,
    ],
)(_layer_body)


def _mean_body(a, b, c, d, o):
    o[...] = (a[...] + b[...] + c[...] + d[...]) * 0.25


def _mean4(e0, e1, e2, e3):
    bs = pl.BlockSpec((1024, _D), lambda i: (i, 0))
    return pl.pallas_call(
        _mean_body,
        grid=(_NPAD // 1024,),
        in_specs=[bs] * 4,
        out_specs=bs,
        out_shape=jax.ShapeDtypeStruct((_NPAD, _D), jnp.float32),
    )(e0, e1, e2, e3)


def kernel(all_users, all_items, edge_index, edge_weight):
    n_users = all_users.shape[0]
    emb0 = jnp.concatenate([all_users, all_items], axis=0)
    emb0 = jnp.pad(emb0, ((0, _NPAD - _N), (0, 0)))
    src = edge_index[0].astype(jnp.int32)
    dst = edge_index[1].astype(jnp.int32)
    w = edge_weight.astype(jnp.float32)
    pad = _EPAD - src.shape[0]
    src = jnp.pad(src, (0, pad)).reshape(_IDXROWS, 128)
    dst = jnp.pad(dst, (0, pad), constant_values=_NPAD).reshape(_IDXROWS, 128)
    w = jnp.pad(w, (0, pad))
    e1 = _layer(emb0, src, dst, w)
    e2 = _layer(e1, src, dst, w)
    e3 = _layer(e2, src, dst, w)
    out = _mean4(emb0, e1, e2, e3)
    return (out[:n_users], out[n_users:_N])
